# baseline (speedup 1.0000x reference)
import functools

import jax
import jax.numpy as jnp
from jax import lax
from jax.experimental import pallas as pl
from jax.experimental.pallas import tpu as pltpu
from jax.experimental.pallas import tpu_sc as plsc

NC = 2
NS = 16
NW = NC * NS
L = 16

EC = 1280
KC = 1568
PC = 3920
BLK = 65536

_SC_PARAMS = dict(
    compiler_params=pltpu.CompilerParams(
        needs_layout_passes=False, use_tc_tiling_on_sc=False))


def _sc_mesh():
    return plsc.VectorSubcoreMesh(core_axis_name="c", subcore_axis_name="s",
                                  num_cores=NC, num_subcores=NS)


def _build_pack(n_nodes):
    n_full = n_nodes // KC
    tail = n_nodes - n_full * KC
    assert tail % L == 0

    @functools.partial(
        pl.kernel,
        out_type=jax.ShapeDtypeStruct(((n_nodes + 1) * 8,), jnp.float32),
        mesh=_sc_mesh(),
        scratch_types=[
            pltpu.VMEM((KC,), jnp.float32),
            pltpu.VMEM((KC,), jnp.float32),
            pltpu.VMEM((KC,), jnp.float32),
            pltpu.VMEM((KC,), jnp.float32),
            pltpu.VMEM((KC * 8,), jnp.float32),
        ],
        **_SC_PARAMS,
    )
    def pack(coord_t_hbm, nv_hbm, node_hbm, cx, cy, v0, v1, node_v):
        wid = lax.axis_index("s") * NC + lax.axis_index("c")
        iota = lax.iota(jnp.int32, L)

        def do_chunk(base, n):
            sl = pl.ds(0, n)
            pltpu.sync_copy(coord_t_hbm.at[0, pl.ds(base, n)], cx.at[sl])
            pltpu.sync_copy(coord_t_hbm.at[1, pl.ds(base, n)], cy.at[sl])
            pltpu.sync_copy(nv_hbm.at[0, pl.ds(base, n)], v0.at[sl])
            pltpu.sync_copy(nv_hbm.at[1, pl.ds(base, n)], v1.at[sl])

            @pl.loop(0, n // L)
            def _ilv(g):
                lane = iota + g * L
                s = pl.ds(g * L, L)
                lane8 = lane * 8
                plsc.store_scatter(node_v, [lane8], cx[s])
                plsc.store_scatter(node_v, [lane8 + 1], cy[s])
                plsc.store_scatter(node_v, [lane8 + 2], v0[s])
                plsc.store_scatter(node_v, [lane8 + 3], v1[s])

            pltpu.sync_copy(node_v.at[pl.ds(0, n * 8)],
                            node_hbm.at[pl.ds((base + 1) * 8, n * 8)])

        @pl.loop(wid, n_full, step=NW)
        def _main(c):
            do_chunk(c * KC, KC)

        if tail:
            @pl.when(wid == n_full % NW)
            def _tail():
                do_chunk(n_full * KC, tail)

    return pack


def _build_stage_a(n_elem):
    n_full = n_elem // EC
    tail = n_elem - n_full * EC
    tail_worker = n_full % NW
    assert tail % L == 0

    @functools.partial(
        pl.kernel,
        out_type=jax.ShapeDtypeStruct((n_elem * 8,), jnp.float32),
        mesh=_sc_mesh(),
        scratch_types=[
            pltpu.VMEM((3 * EC,), jnp.int32),
            pltpu.VMEM((3 * EC,), jnp.int32),
            pltpu.VMEM((3 * EC, 8), jnp.float32),
            pltpu.VMEM((3 * EC, 8), jnp.float32),
            pltpu.VMEM((EC * 8,), jnp.float32),
            pltpu.VMEM((EC * 8,), jnp.float32),
            pltpu.SemaphoreType.DMA,
            pltpu.SemaphoreType.DMA,
            pltpu.SemaphoreType.DMA,
            pltpu.SemaphoreType.DMA,
            pltpu.SemaphoreType.DMA,
            pltpu.SemaphoreType.DMA,
        ],
        **_SC_PARAMS,
    )
    def stage_a(conn_t_hbm, node_hbm, m_hbm,
                idx_a, idx_b, rows_a, rows_b, m_a, m_b,
                isem_a, isem_b, gsem_a, gsem_b, wsem_a, wsem_b):
        wid = lax.axis_index("s") * NC + lax.axis_index("c")
        iota = lax.iota(jnp.int32, L)
        idx_ = (idx_a, idx_b)
        rows_ = (rows_a, rows_b)
        m_ = (m_a, m_b)
        isem_ = (isem_a, isem_b)
        gsem_ = (gsem_a, gsem_b)
        wsem_ = (wsem_a, wsem_b)

        def issue_idx(c, b):
            for k in range(3):
                pltpu.async_copy(conn_t_hbm.at[k, pl.ds(c * EC, EC)],
                                 idx_[b].at[pl.ds(k * EC, EC)], isem_[b])

        def issue_gather(b):
            for k in range(3):
                pltpu.make_async_copy(conn_t_hbm.at[k, pl.ds(0, EC)],
                                      idx_[b].at[pl.ds(k * EC, EC)],
                                      isem_[b]).wait()
            pltpu.async_copy(node_hbm.at[idx_[b]], rows_[b], gsem_[b])

        def wait_gather(b):
            pltpu.make_async_copy(node_hbm.at[idx_[b]], rows_[b],
                                  gsem_[b]).wait()

        def issue_wb(c, b):
            pltpu.async_copy(m_[b], m_hbm.at[pl.ds(c * EC * 8, EC * 8)],
                             wsem_[b])

        def wait_wb(b):
            pltpu.make_async_copy(m_[b], m_hbm.at[pl.ds(0, EC * 8)],
                                  wsem_[b]).wait()

        def compute(b, n):
            rows = rows_[b]
            m_v = m_[b]

            @pl.loop(0, n // L)
            def _compute(g):
                lane = iota + g * L
                z = jnp.zeros((L,), jnp.int32)

                def fld(k, c):
                    return plsc.load_gather(rows, [lane + k * EC, z + c])

                x1 = fld(0, 0); y1 = fld(0, 1); u1 = fld(0, 2); w1 = fld(0, 3)
                x2 = fld(1, 0); y2 = fld(1, 1); u2 = fld(1, 2); w2 = fld(1, 3)
                x3 = fld(2, 0); y3 = fld(2, 1); u3 = fld(2, 2); w3 = fld(2, 3)

                d1 = x1 * (y3 - y2) + x2 * (y1 - y3) + x3 * (y2 - y1)
                d2 = -x1 * y2 + x1 * y3 + x2 * y1 - x2 * y3 - x3 * y1 + x3 * y2
                d3 = x1 * (y2 - y3) + x2 * (y3 - y1) + x3 * (y1 - y2)
                m00 = (y3 - y2) / d1
                m10 = (x2 - x3) / d2
                m20 = (x3 * y2 - x2 * y3) / d2
                m01 = (y1 - y3) / d2
                m11 = (x1 - x3) / d3
                m21 = (x3 * y1 - x1 * y3) / d3
                m02 = (y1 - y2) / d3
                m12 = (x1 - x2) / d2
                m22 = (x2 * y1 - x1 * y2) / d2

                a0 = m00 * u1 + m01 * u2 + m02 * u3
                b0 = m10 * u1 + m11 * u2 + m12 * u3
                c0 = m20 * u1 + m21 * u2 + m22 * u3
                a1 = m00 * w1 + m01 * w2 + m02 * w3
                b1 = m10 * w1 + m11 * w2 + m12 * w3
                c1 = m20 * w1 + m21 * w2 + m22 * w3

                lane8 = lane * 8
                plsc.store_scatter(m_v, [lane8], a0)
                plsc.store_scatter(m_v, [lane8 + 1], b0)
                plsc.store_scatter(m_v, [lane8 + 2], c0)
                plsc.store_scatter(m_v, [lane8 + 3], a1)
                plsc.store_scatter(m_v, [lane8 + 4], b1)
                plsc.store_scatter(m_v, [lane8 + 5], c1)

        @pl.when(wid < n_full)
        def _p0():
            issue_idx(wid, 0)

        @pl.when(wid + NW < n_full)
        def _p1():
            issue_idx(wid + NW, 1)

        @pl.when(wid < n_full)
        def _p2():
            issue_gather(0)

        @pl.loop(wid, n_full, step=2 * NW)
        def _main(c0):
            c1 = c0 + NW
            c2 = c0 + 2 * NW
            c3 = c0 + 3 * NW

            @pl.when(c1 < n_full)
            def _g1():
                issue_gather(1)

            wait_gather(0)

            @pl.when(c2 < n_full)
            def _i0():
                issue_idx(c2, 0)

            @pl.when(c0 > wid)
            def _w0():
                wait_wb(0)

            compute(0, EC)
            issue_wb(c0, 0)

            @pl.when(c1 < n_full)
            def _s1():
                @pl.when(c2 < n_full)
                def _g0():
                    issue_gather(0)

                wait_gather(1)

                @pl.when(c3 < n_full)
                def _i1():
                    issue_idx(c3, 1)

                @pl.when(c1 > wid + NW)
                def _w1():
                    wait_wb(1)

                compute(1, EC)
                issue_wb(c1, 1)

        @pl.when(wid < n_full)
        def _d0():
            wait_wb(0)

        @pl.when(wid + NW < n_full)
        def _d1():
            wait_wb(1)

        if tail:
            @pl.when(wid == tail_worker)
            def _tail():
                base = n_full * EC
                for k in range(3):
                    pltpu.sync_copy(conn_t_hbm.at[k, pl.ds(base, tail)],
                                    idx_[0].at[pl.ds(k * EC, tail)])
                cps = [pltpu.async_copy(
                           node_hbm.at[idx_[0].at[pl.ds(k * EC, tail)]],
                           rows_[0].at[pl.ds(k * EC, tail), :], gsem_[0])
                       for k in range(3)]
                for cp in cps:
                    cp.wait()
                compute(0, tail)
                pltpu.sync_copy(m_[0].at[pl.ds(0, tail * 8)],
                                m_hbm.at[pl.ds(base * 8, tail * 8)])

    return stage_a


def _build_stage_b(start, count):
    n_full = count // PC
    tail = count - n_full * PC
    tail_worker = n_full % NW
    assert tail % L == 0 and start % 8 == 0
    plane = jax.ShapeDtypeStruct((count,), jnp.float32)

    @functools.partial(
        pl.kernel,
        out_type=(plane,) * 6,
        mesh=_sc_mesh(),
        scratch_types=[
            pltpu.VMEM((PC,), jnp.int32),
            pltpu.VMEM((PC,), jnp.int32),
            pltpu.VMEM((PC, 8), jnp.float32),
            pltpu.VMEM((PC, 8), jnp.float32),
            pltpu.VMEM((PC,), jnp.float32),
            pltpu.VMEM((PC,), jnp.float32),
            pltpu.VMEM((PC,), jnp.float32),
            pltpu.VMEM((PC,), jnp.float32),
            pltpu.VMEM((PC,), jnp.float32),
            pltpu.VMEM((PC,), jnp.float32),
            pltpu.VMEM((PC,), jnp.float32),
            pltpu.VMEM((PC,), jnp.float32),
            pltpu.VMEM((PC,), jnp.float32),
            pltpu.VMEM((PC,), jnp.float32),
            pltpu.VMEM((PC,), jnp.float32),
            pltpu.VMEM((PC,), jnp.float32),
            pltpu.SemaphoreType.DMA,
            pltpu.SemaphoreType.DMA,
            pltpu.SemaphoreType.DMA,
            pltpu.SemaphoreType.DMA,
            pltpu.SemaphoreType.DMA,
            pltpu.SemaphoreType.DMA,
        ],
        **_SC_PARAMS,
    )
    def stage_b(cid_hbm, m_hbm, oa0, ob0, oc0, oa1, ob1, oc1,
                cid_a, cid_b, rows_a, rows_b,
                p00, p01, p02, p03, p04, p05,
                p10, p11, p12, p13, p14, p15,
                csem_a, csem_b, gsem_a, gsem_b, wsem_a, wsem_b):
        wid = lax.axis_index("s") * NC + lax.axis_index("c")
        iota = lax.iota(jnp.int32, L)
        cid_ = (cid_a, cid_b)
        rows_ = (rows_a, rows_b)
        pl_ = ((p00, p01, p02, p03, p04, p05),
               (p10, p11, p12, p13, p14, p15))
        out_ = (oa0, ob0, oc0, oa1, ob1, oc1)
        csem_ = (csem_a, csem_b)
        gsem_ = (gsem_a, gsem_b)
        wsem_ = (wsem_a, wsem_b)

        def issue_cid(c, b):
            pltpu.async_copy(cid_hbm.at[pl.ds(start + c * PC, PC)], cid_[b],
                             csem_[b])

        def issue_gather(b):
            pltpu.make_async_copy(cid_hbm.at[pl.ds(0, PC)], cid_[b],
                                  csem_[b]).wait()
            pltpu.async_copy(m_hbm.at[cid_[b]], rows_[b], gsem_[b])

        def wait_gather(b):
            pltpu.make_async_copy(m_hbm.at[cid_[b]], rows_[b],
                                  gsem_[b]).wait()

        def issue_wb(c, b):
            for j in range(6):
                pltpu.async_copy(pl_[b][j], out_[j].at[pl.ds(c * PC, PC)],
                                 wsem_[b])

        def wait_wb(b):
            for j in range(6):
                pltpu.make_async_copy(pl_[b][j], out_[j].at[pl.ds(0, PC)],
                                      wsem_[b]).wait()

        def compute(b, n):
            rows = rows_[b]
            planes = pl_[b]

            @pl.loop(0, n // L)
            def _cmp(g):
                lane = iota + g * L
                z = jnp.zeros((L,), jnp.int32)
                s = pl.ds(g * L, L)
                planes[0][s] = plsc.load_gather(rows, [lane, z])
                planes[1][s] = plsc.load_gather(rows, [lane, z + 1])
                planes[2][s] = plsc.load_gather(rows, [lane, z + 2])
                planes[3][s] = plsc.load_gather(rows, [lane, z + 3])
                planes[4][s] = plsc.load_gather(rows, [lane, z + 4])
                planes[5][s] = plsc.load_gather(rows, [lane, z + 5])

        @pl.when(wid < n_full)
        def _p0():
            issue_cid(wid, 0)

        @pl.when(wid + NW < n_full)
        def _p1():
            issue_cid(wid + NW, 1)

        @pl.when(wid < n_full)
        def _p2():
            issue_gather(0)

        @pl.loop(wid, n_full, step=2 * NW)
        def _main(c0):
            c1 = c0 + NW
            c2 = c0 + 2 * NW
            c3 = c0 + 3 * NW

            @pl.when(c1 < n_full)
            def _g1():
                issue_gather(1)

            wait_gather(0)

            @pl.when(c2 < n_full)
            def _i0():
                issue_cid(c2, 0)

            @pl.when(c0 > wid)
            def _w0():
                wait_wb(0)

            compute(0, PC)
            issue_wb(c0, 0)

            @pl.when(c1 < n_full)
            def _s1():
                @pl.when(c2 < n_full)
                def _g0():
                    issue_gather(0)

                wait_gather(1)

                @pl.when(c3 < n_full)
                def _i1():
                    issue_cid(c3, 1)

                @pl.when(c1 > wid + NW)
                def _w1():
                    wait_wb(1)

                compute(1, PC)
                issue_wb(c1, 1)

        @pl.when(wid < n_full)
        def _d0():
            wait_wb(0)

        @pl.when(wid + NW < n_full)
        def _d1():
            wait_wb(1)

        if tail:
            @pl.when(wid == tail_worker)
            def _tail():
                base = n_full * PC
                sl = pl.ds(0, tail)
                pltpu.sync_copy(cid_hbm.at[pl.ds(start + base, tail)],
                                cid_[0].at[sl])
                pltpu.async_copy(m_hbm.at[cid_[0].at[sl]],
                                 rows_[0].at[sl, :], gsem_[0]).wait()
                compute(0, tail)
                for j in range(6):
                    pltpu.sync_copy(pl_[0][j].at[sl],
                                    out_[j].at[pl.ds(base, tail)])

    return stage_b


def _build_stage_c(n_pts, start, count, aliased):
    blk0 = start // BLK

    def xmap(i):
        return (0, blk0 + i)

    pspec = pl.BlockSpec((BLK,), lambda i: (i,))
    xspec = pl.BlockSpec((2, BLK), xmap)

    if aliased:
        def body(prev_ref, xt_ref, a0, b0, c0, a1, b1, c1, out_ref):
            xx = xt_ref[0, :]
            yy = xt_ref[1, :]
            out_ref[0, :] = xx * a0[:] + yy * b0[:] + c0[:]
            out_ref[1, :] = xx * a1[:] + yy * b1[:] + c1[:]

        return pl.pallas_call(
            body,
            grid=(pl.cdiv(count, BLK),),
            in_specs=[xspec, xspec] + [pspec] * 6,
            out_specs=xspec,
            out_shape=jax.ShapeDtypeStruct((2, n_pts), jnp.float32),
            input_output_aliases={0: 0},
        )

    def body(xt_ref, a0, b0, c0, a1, b1, c1, out_ref):
        xx = xt_ref[0, :]
        yy = xt_ref[1, :]
        out_ref[0, :] = xx * a0[:] + yy * b0[:] + c0[:]
        out_ref[1, :] = xx * a1[:] + yy * b1[:] + c1[:]

    return pl.pallas_call(
        body,
        grid=(pl.cdiv(count, BLK),),
        in_specs=[xspec] + [pspec] * 6,
        out_specs=xspec,
        out_shape=jax.ShapeDtypeStruct((2, n_pts), jnp.float32),
    )


def kernel(x, coordinates, nodal_values, connectivity, cell_id):
    n_pts = x.shape[0]
    n_elem = connectivity.shape[0]
    n_nodes = coordinates.shape[0]

    coord_t = coordinates.T
    conn_t = connectivity.astype(jnp.int32).T
    xt = x.T

    node_flat = _build_pack(n_nodes)(coord_t, nodal_values.astype(jnp.float32))
    node_tab = node_flat.reshape(n_nodes + 1, 8)

    m_flat = _build_stage_a(n_elem)(conn_t, node_tab)
    m_tab = m_flat.reshape(n_elem, 8)

    cid = cell_id.astype(jnp.int32)
    half_blocks = (n_pts // BLK + 1) // 2
    p1 = half_blocks * BLK
    planes1 = _build_stage_b(0, p1)(cid, m_tab)
    planes2 = _build_stage_b(p1, n_pts - p1)(cid, m_tab)
    out1 = _build_stage_c(n_pts, 0, p1, aliased=False)(xt, *planes1)
    return _build_stage_c(n_pts, p1, n_pts - p1, aliased=True)(
        out1, xt, *planes2)


# speedup vs baseline: 1.1256x; 1.1256x over previous
import functools

import jax
import jax.numpy as jnp
from jax import lax
from jax.experimental import pallas as pl
from jax.experimental.pallas import tpu as pltpu
from jax.experimental.pallas import tpu_sc as plsc

NC = 2
NS = 16
NW = NC * NS
L = 16

EC = 1280
KC = 1568
PC = 3920
BLK = 65536

_SC_PARAMS = dict(
    compiler_params=pltpu.CompilerParams(
        needs_layout_passes=False, use_tc_tiling_on_sc=False))


def _sc_mesh():
    return plsc.VectorSubcoreMesh(core_axis_name="c", subcore_axis_name="s",
                                  num_cores=NC, num_subcores=NS)


def _build_pack(n_nodes):
    n_full = n_nodes // KC
    tail = n_nodes - n_full * KC
    assert tail % L == 0

    @functools.partial(
        pl.kernel,
        out_type=jax.ShapeDtypeStruct(((n_nodes + 1) * 8,), jnp.float32),
        mesh=_sc_mesh(),
        scratch_types=[
            pltpu.VMEM((KC,), jnp.float32),
            pltpu.VMEM((KC,), jnp.float32),
            pltpu.VMEM((KC,), jnp.float32),
            pltpu.VMEM((KC,), jnp.float32),
            pltpu.VMEM((KC * 8,), jnp.float32),
        ],
        **_SC_PARAMS,
    )
    def pack(coord_t_hbm, nv_hbm, node_hbm, cx, cy, v0, v1, node_v):
        wid = lax.axis_index("s") * NC + lax.axis_index("c")
        iota = lax.iota(jnp.int32, L)

        def do_chunk(base, n):
            sl = pl.ds(0, n)
            pltpu.sync_copy(coord_t_hbm.at[0, pl.ds(base, n)], cx.at[sl])
            pltpu.sync_copy(coord_t_hbm.at[1, pl.ds(base, n)], cy.at[sl])
            pltpu.sync_copy(nv_hbm.at[0, pl.ds(base, n)], v0.at[sl])
            pltpu.sync_copy(nv_hbm.at[1, pl.ds(base, n)], v1.at[sl])

            @pl.loop(0, n // L)
            def _ilv(g):
                lane = iota + g * L
                s = pl.ds(g * L, L)
                lane8 = lane * 8
                plsc.store_scatter(node_v, [lane8], cx[s])
                plsc.store_scatter(node_v, [lane8 + 1], cy[s])
                plsc.store_scatter(node_v, [lane8 + 2], v0[s])
                plsc.store_scatter(node_v, [lane8 + 3], v1[s])

            pltpu.sync_copy(node_v.at[pl.ds(0, n * 8)],
                            node_hbm.at[pl.ds((base + 1) * 8, n * 8)])

        @pl.loop(wid, n_full, step=NW)
        def _main(c):
            do_chunk(c * KC, KC)

        if tail:
            @pl.when(wid == n_full % NW)
            def _tail():
                do_chunk(n_full * KC, tail)

    return pack


def _build_stage_a(n_elem):
    n_full = n_elem // EC
    tail = n_elem - n_full * EC
    tail_worker = n_full % NW
    assert tail % L == 0

    @functools.partial(
        pl.kernel,
        out_type=jax.ShapeDtypeStruct((n_elem * 8,), jnp.float32),
        mesh=_sc_mesh(),
        scratch_types=[
            pltpu.VMEM((3 * EC,), jnp.int32),
            pltpu.VMEM((3 * EC,), jnp.int32),
            pltpu.VMEM((3 * EC, 8), jnp.float32),
            pltpu.VMEM((3 * EC, 8), jnp.float32),
            pltpu.VMEM((EC * 8,), jnp.float32),
            pltpu.VMEM((EC * 8,), jnp.float32),
            pltpu.SemaphoreType.DMA,
            pltpu.SemaphoreType.DMA,
            pltpu.SemaphoreType.DMA,
            pltpu.SemaphoreType.DMA,
            pltpu.SemaphoreType.DMA,
            pltpu.SemaphoreType.DMA,
        ],
        **_SC_PARAMS,
    )
    def stage_a(conn_t_hbm, node_hbm, m_hbm,
                idx_a, idx_b, rows_a, rows_b, m_a, m_b,
                isem_a, isem_b, gsem_a, gsem_b, wsem_a, wsem_b):
        wid = lax.axis_index("s") * NC + lax.axis_index("c")
        iota = lax.iota(jnp.int32, L)
        idx_ = (idx_a, idx_b)
        rows_ = (rows_a, rows_b)
        m_ = (m_a, m_b)
        isem_ = (isem_a, isem_b)
        gsem_ = (gsem_a, gsem_b)
        wsem_ = (wsem_a, wsem_b)

        def issue_idx(c, b):
            for k in range(3):
                pltpu.async_copy(conn_t_hbm.at[k, pl.ds(c * EC, EC)],
                                 idx_[b].at[pl.ds(k * EC, EC)], isem_[b])

        def issue_gather(b):
            for k in range(3):
                pltpu.make_async_copy(conn_t_hbm.at[k, pl.ds(0, EC)],
                                      idx_[b].at[pl.ds(k * EC, EC)],
                                      isem_[b]).wait()
            pltpu.async_copy(node_hbm.at[idx_[b]], rows_[b], gsem_[b])

        def wait_gather(b):
            pltpu.make_async_copy(node_hbm.at[idx_[b]], rows_[b],
                                  gsem_[b]).wait()

        def issue_wb(c, b):
            pltpu.async_copy(m_[b], m_hbm.at[pl.ds(c * EC * 8, EC * 8)],
                             wsem_[b])

        def wait_wb(b):
            pltpu.make_async_copy(m_[b], m_hbm.at[pl.ds(0, EC * 8)],
                                  wsem_[b]).wait()

        def compute(b, n):
            rows = rows_[b]
            m_v = m_[b]

            @pl.loop(0, n // L)
            def _compute(g):
                lane = iota + g * L
                z = jnp.zeros((L,), jnp.int32)

                def fld(k, c):
                    return plsc.load_gather(rows, [lane + k * EC, z + c])

                x1 = fld(0, 0); y1 = fld(0, 1); u1 = fld(0, 2); w1 = fld(0, 3)
                x2 = fld(1, 0); y2 = fld(1, 1); u2 = fld(1, 2); w2 = fld(1, 3)
                x3 = fld(2, 0); y3 = fld(2, 1); u3 = fld(2, 2); w3 = fld(2, 3)

                d1 = x1 * (y3 - y2) + x2 * (y1 - y3) + x3 * (y2 - y1)
                d2 = -x1 * y2 + x1 * y3 + x2 * y1 - x2 * y3 - x3 * y1 + x3 * y2
                d3 = x1 * (y2 - y3) + x2 * (y3 - y1) + x3 * (y1 - y2)
                m00 = (y3 - y2) / d1
                m10 = (x2 - x3) / d2
                m20 = (x3 * y2 - x2 * y3) / d2
                m01 = (y1 - y3) / d2
                m11 = (x1 - x3) / d3
                m21 = (x3 * y1 - x1 * y3) / d3
                m02 = (y1 - y2) / d3
                m12 = (x1 - x2) / d2
                m22 = (x2 * y1 - x1 * y2) / d2

                a0 = m00 * u1 + m01 * u2 + m02 * u3
                b0 = m10 * u1 + m11 * u2 + m12 * u3
                c0 = m20 * u1 + m21 * u2 + m22 * u3
                a1 = m00 * w1 + m01 * w2 + m02 * w3
                b1 = m10 * w1 + m11 * w2 + m12 * w3
                c1 = m20 * w1 + m21 * w2 + m22 * w3

                lane8 = lane * 8
                plsc.store_scatter(m_v, [lane8], a0)
                plsc.store_scatter(m_v, [lane8 + 1], b0)
                plsc.store_scatter(m_v, [lane8 + 2], c0)
                plsc.store_scatter(m_v, [lane8 + 3], a1)
                plsc.store_scatter(m_v, [lane8 + 4], b1)
                plsc.store_scatter(m_v, [lane8 + 5], c1)

        @pl.when(wid < n_full)
        def _p0():
            issue_idx(wid, 0)

        @pl.when(wid + NW < n_full)
        def _p1():
            issue_idx(wid + NW, 1)

        @pl.when(wid < n_full)
        def _p2():
            issue_gather(0)

        @pl.loop(wid, n_full, step=2 * NW)
        def _main(c0):
            c1 = c0 + NW
            c2 = c0 + 2 * NW
            c3 = c0 + 3 * NW

            @pl.when(c1 < n_full)
            def _g1():
                issue_gather(1)

            wait_gather(0)

            @pl.when(c2 < n_full)
            def _i0():
                issue_idx(c2, 0)

            @pl.when(c0 > wid)
            def _w0():
                wait_wb(0)

            compute(0, EC)
            issue_wb(c0, 0)

            @pl.when(c1 < n_full)
            def _s1():
                @pl.when(c2 < n_full)
                def _g0():
                    issue_gather(0)

                wait_gather(1)

                @pl.when(c3 < n_full)
                def _i1():
                    issue_idx(c3, 1)

                @pl.when(c1 > wid + NW)
                def _w1():
                    wait_wb(1)

                compute(1, EC)
                issue_wb(c1, 1)

        @pl.when(wid < n_full)
        def _d0():
            wait_wb(0)

        @pl.when(wid + NW < n_full)
        def _d1():
            wait_wb(1)

        if tail:
            @pl.when(wid == tail_worker)
            def _tail():
                base = n_full * EC
                for k in range(3):
                    pltpu.sync_copy(conn_t_hbm.at[k, pl.ds(base, tail)],
                                    idx_[0].at[pl.ds(k * EC, tail)])
                cps = [pltpu.async_copy(
                           node_hbm.at[idx_[0].at[pl.ds(k * EC, tail)]],
                           rows_[0].at[pl.ds(k * EC, tail), :], gsem_[0])
                       for k in range(3)]
                for cp in cps:
                    cp.wait()
                compute(0, tail)
                pltpu.sync_copy(m_[0].at[pl.ds(0, tail * 8)],
                                m_hbm.at[pl.ds(base * 8, tail * 8)])

    return stage_a


def _build_stage_b(n_pts):
    n_full = n_pts // PC
    tail = n_pts - n_full * PC
    tail_worker = n_full % NW
    assert tail % L == 0
    plane = jax.ShapeDtypeStruct((n_pts,), jnp.float32)

    @functools.partial(
        pl.kernel,
        out_type=(plane,) * 6,
        mesh=_sc_mesh(),
        scratch_types=[
            pltpu.VMEM((PC,), jnp.int32),
            pltpu.VMEM((PC,), jnp.int32),
            pltpu.VMEM((PC, 8), jnp.float32),
            pltpu.VMEM((PC, 8), jnp.float32),
            pltpu.VMEM((PC,), jnp.float32),
            pltpu.VMEM((PC,), jnp.float32),
            pltpu.VMEM((PC,), jnp.float32),
            pltpu.VMEM((PC,), jnp.float32),
            pltpu.VMEM((PC,), jnp.float32),
            pltpu.VMEM((PC,), jnp.float32),
            pltpu.VMEM((PC,), jnp.float32),
            pltpu.VMEM((PC,), jnp.float32),
            pltpu.VMEM((PC,), jnp.float32),
            pltpu.VMEM((PC,), jnp.float32),
            pltpu.VMEM((PC,), jnp.float32),
            pltpu.VMEM((PC,), jnp.float32),
            pltpu.SemaphoreType.DMA,
            pltpu.SemaphoreType.DMA,
            pltpu.SemaphoreType.DMA,
            pltpu.SemaphoreType.DMA,
            pltpu.SemaphoreType.DMA,
            pltpu.SemaphoreType.DMA,
        ],
        **_SC_PARAMS,
    )
    def stage_b(cid_hbm, m_hbm, oa0, ob0, oc0, oa1, ob1, oc1,
                cid_a, cid_b, rows_a, rows_b,
                p00, p01, p02, p03, p04, p05,
                p10, p11, p12, p13, p14, p15,
                csem_a, csem_b, gsem_a, gsem_b, wsem_a, wsem_b):
        wid = lax.axis_index("s") * NC + lax.axis_index("c")
        iota = lax.iota(jnp.int32, L)
        cid_ = (cid_a, cid_b)
        rows_ = (rows_a, rows_b)
        pl_ = ((p00, p01, p02, p03, p04, p05),
               (p10, p11, p12, p13, p14, p15))
        out_ = (oa0, ob0, oc0, oa1, ob1, oc1)
        csem_ = (csem_a, csem_b)
        gsem_ = (gsem_a, gsem_b)
        wsem_ = (wsem_a, wsem_b)

        def issue_cid(c, b):
            pltpu.async_copy(cid_hbm.at[pl.ds(c * PC, PC)], cid_[b], csem_[b])

        def issue_gather(b):
            pltpu.make_async_copy(cid_hbm.at[pl.ds(0, PC)], cid_[b],
                                  csem_[b]).wait()
            pltpu.async_copy(m_hbm.at[cid_[b]], rows_[b], gsem_[b])

        def wait_gather(b):
            pltpu.make_async_copy(m_hbm.at[cid_[b]], rows_[b],
                                  gsem_[b]).wait()

        def issue_wb(c, b):
            for j in range(6):
                pltpu.async_copy(pl_[b][j], out_[j].at[pl.ds(c * PC, PC)],
                                 wsem_[b])

        def wait_wb(b):
            for j in range(6):
                pltpu.make_async_copy(pl_[b][j], out_[j].at[pl.ds(0, PC)],
                                      wsem_[b]).wait()

        def compute(b, n):
            rows = rows_[b]
            planes = pl_[b]

            @pl.loop(0, n // L)
            def _cmp(g):
                lane = iota + g * L
                z = jnp.zeros((L,), jnp.int32)
                s = pl.ds(g * L, L)
                planes[0][s] = plsc.load_gather(rows, [lane, z])
                planes[1][s] = plsc.load_gather(rows, [lane, z + 1])
                planes[2][s] = plsc.load_gather(rows, [lane, z + 2])
                planes[3][s] = plsc.load_gather(rows, [lane, z + 3])
                planes[4][s] = plsc.load_gather(rows, [lane, z + 4])
                planes[5][s] = plsc.load_gather(rows, [lane, z + 5])

        @pl.when(wid < n_full)
        def _p0():
            issue_cid(wid, 0)

        @pl.when(wid + NW < n_full)
        def _p1():
            issue_cid(wid + NW, 1)

        @pl.when(wid < n_full)
        def _p2():
            issue_gather(0)

        @pl.loop(wid, n_full, step=2 * NW)
        def _main(c0):
            c1 = c0 + NW
            c2 = c0 + 2 * NW
            c3 = c0 + 3 * NW

            @pl.when(c1 < n_full)
            def _g1():
                issue_gather(1)

            wait_gather(0)

            @pl.when(c2 < n_full)
            def _i0():
                issue_cid(c2, 0)

            @pl.when(c0 > wid)
            def _w0():
                wait_wb(0)

            compute(0, PC)
            issue_wb(c0, 0)

            @pl.when(c1 < n_full)
            def _s1():
                @pl.when(c2 < n_full)
                def _g0():
                    issue_gather(0)

                wait_gather(1)

                @pl.when(c3 < n_full)
                def _i1():
                    issue_cid(c3, 1)

                @pl.when(c1 > wid + NW)
                def _w1():
                    wait_wb(1)

                compute(1, PC)
                issue_wb(c1, 1)

        @pl.when(wid < n_full)
        def _d0():
            wait_wb(0)

        @pl.when(wid + NW < n_full)
        def _d1():
            wait_wb(1)

        if tail:
            @pl.when(wid == tail_worker)
            def _tail():
                base = n_full * PC
                sl = pl.ds(0, tail)
                pltpu.sync_copy(cid_hbm.at[pl.ds(base, tail)], cid_[0].at[sl])
                pltpu.async_copy(m_hbm.at[cid_[0].at[sl]],
                                 rows_[0].at[sl, :], gsem_[0]).wait()
                compute(0, tail)
                for j in range(6):
                    pltpu.sync_copy(pl_[0][j].at[sl],
                                    out_[j].at[pl.ds(base, tail)])

    return stage_b


def _build_stage_c(n_pts):
    def body(xt_ref, a0, b0, c0, a1, b1, c1, out_ref):
        xx = xt_ref[0, :]
        yy = xt_ref[1, :]
        out_ref[0, :] = xx * a0[:] + yy * b0[:] + c0[:]
        out_ref[1, :] = xx * a1[:] + yy * b1[:] + c1[:]

    pspec = pl.BlockSpec((BLK,), lambda i: (i,))
    return pl.pallas_call(
        body,
        grid=(pl.cdiv(n_pts, BLK),),
        in_specs=[pl.BlockSpec((2, BLK), lambda i: (0, i))] + [pspec] * 6,
        out_specs=pl.BlockSpec((2, BLK), lambda i: (0, i)),
        out_shape=jax.ShapeDtypeStruct((2, n_pts), jnp.float32),
    )


def kernel(x, coordinates, nodal_values, connectivity, cell_id):
    n_pts = x.shape[0]
    n_elem = connectivity.shape[0]
    n_nodes = coordinates.shape[0]

    coord_t = coordinates.T
    conn_t = connectivity.astype(jnp.int32).T
    xt = x.T

    node_flat = _build_pack(n_nodes)(coord_t, nodal_values.astype(jnp.float32))
    node_tab = node_flat.reshape(n_nodes + 1, 8)

    m_flat = _build_stage_a(n_elem)(conn_t, node_tab)
    m_tab = m_flat.reshape(n_elem, 8)

    planes = _build_stage_b(n_pts)(cell_id.astype(jnp.int32), m_tab)
    return _build_stage_c(n_pts)(xt, *planes)
